# baseline (speedup 1.0000x reference)
import functools

import jax
import jax.numpy as jnp
from jax import lax
from jax.experimental import pallas as pl
from jax.experimental.pallas import tpu as pltpu
from jax.experimental.pallas import tpu_sc as plsc

_WIN = 4
_LANE = 128
_L = 16
_NSUB = 32
_NBIN = 256
_NSEC = 8


def _temporal_kernel(x_ref, t_ref, out_ref, acc_ref):
    step = pl.program_id(0)
    x = x_ref[0]
    tg = t_ref[0]

    @pl.when(step == 0)
    def _():
        acc_ref[0] = 0.0

    mo = jnp.maximum(jnp.maximum(x[0], x[1]), jnp.maximum(x[2], x[3]))
    mt = jnp.maximum(jnp.maximum(tg[0], tg[1]), jnp.maximum(tg[2], tg[3]))
    dt = mo - mt
    acc_ref[0] = acc_ref[0] + jnp.sum(dt * dt)

    @pl.when(step == pl.num_programs(0) - 1)
    def _():
        out_ref[0, 0] = acc_ref[0]


def _temporal_sse(output, target):
    B, T, C, H, W = output.shape
    hw = H * W
    sub = hw // _LANE
    nw = T // _WIN
    xr = output.reshape(B * nw, _WIN, C, sub, _LANE)
    tr = target.reshape(B * nw, _WIN, C, sub, _LANE)
    spec = pl.BlockSpec((1, _WIN, C, sub, _LANE),
                        lambda r: (r, 0, 0, 0, 0))
    out = pl.pallas_call(
        _temporal_kernel,
        grid=(B * nw,),
        in_specs=[spec, spec],
        out_specs=pl.BlockSpec(memory_space=pltpu.SMEM),
        out_shape=jax.ShapeDtypeStruct((1, 1), jnp.float32),
        scratch_shapes=[pltpu.SMEM((1,), jnp.float32)],
    )(xr, tr)
    return out[0, 0]


_NBANK = 1
_NCOPY = _NBANK * _L
_HWORDS = _NBIN * _NCOPY


def _sc_body(nrows, n, kk, rows_per, x_hbm, t_hbm, out_hbm,
             xv, tring, histc, hists, p_ref, ps_ref, outv, sem0, sem1):
    cid = lax.axis_index("c")
    sid = lax.axis_index("s")
    wid = sid * 2 + cid

    iota = lax.iota(jnp.int32, _L)
    lane_base = iota * _NBIN
    ones = jnp.ones((_L,), jnp.int32)
    nchunk = n // _L
    swords = n // _NSEC
    schunk = swords // _L
    i32min = jnp.int32(-2147483648)
    sems = (sem0, sem1)

    def zero_hists():
        @plsc.parallel_loop(0, _HWORDS // _L, unroll=4)
        def _z(i):
            histc[pl.ds(i * _L, _L)] = jnp.zeros((_L,), jnp.int32)
            hists[pl.ds(i * _L, _L)] = jnp.zeros((_L,), jnp.float32)

    def keys_of(v):
        bits = lax.bitcast_convert_type(v, jnp.int32)
        return jnp.where(bits < 0, bits ^ jnp.int32(0x7FFFFFFF), bits)

    def build_p():
        cum = jnp.int32(0)
        cums = jnp.float32(0.0)
        for c in range(_NBIN // _L):
            def mbody(j, acc):
                a, s = acc
                a = a + histc[pl.ds(j * _NBIN + c * _L, _L)]
                s = s + hists[pl.ds(j * _NBIN + c * _L, _L)]
                return a, s
            acc, accs = lax.fori_loop(
                0, _NCOPY, mbody,
                (jnp.zeros((_L,), jnp.int32), jnp.zeros((_L,), jnp.float32)))
            pc = plsc.cumsum(acc) + cum
            psc = plsc.cumsum(accs) + cums
            p_ref[pl.ds(c * _L, _L)] = pc
            ps_ref[pl.ds(c * _L, _L)] = psc
            cum = cum + jnp.sum(acc)
            cums = cums + jnp.sum(accs)
        return cum, cums

    def find_cross(thresh):
        found = jnp.int32(0)
        b_star = jnp.int32(0)
        p_star = jnp.int32(0)
        for c in range(_NBIN // _L):
            pc = p_ref[pl.ds(c * _L, _L)]
            m = pc > thresh
            cand = jnp.where(m, 255 - (iota + c * _L), -1)
            mx = jnp.max(cand)
            bloc = 255 - mx
            pmin = -jnp.max(jnp.where(m, -pc, i32min))
            any_m = mx >= 0
            take = (found == 0) & any_m
            b_star = jnp.where(take, bloc, b_star)
            p_star = jnp.where(take, pmin, p_star)
            found = jnp.where(any_m, jnp.int32(1), found)
        p_prev = jnp.int32(0)
        s_at = jnp.float32(0.0)
        s_prev = jnp.float32(0.0)
        for c in range(_NBIN // _L):
            bins = iota + c * _L
            pc = p_ref[pl.ds(c * _L, _L)]
            psc = ps_ref[pl.ds(c * _L, _L)]
            p_prev = p_prev + jnp.sum(jnp.where(bins == b_star - 1, pc, 0))
            s_at = s_at + jnp.sum(jnp.where(bins == b_star, psc, 0.0))
            s_prev = s_prev + jnp.sum(jnp.where(bins == b_star - 1, psc, 0.0))
        return b_star, p_star, p_prev, s_at, s_prev

    def stream_pass(row, chunk_fn):
        cps = [None, None]
        cps[0] = pltpu.async_copy(
            t_hbm.at[row, pl.ds(0, swords)], tring.at[0], sems[0])
        for s in range(_NSEC):
            par = s % 2
            if s + 1 < _NSEC:
                parn = (s + 1) % 2
                cps[parn] = pltpu.async_copy(
                    t_hbm.at[row, pl.ds((s + 1) * swords, swords)],
                    tring.at[parn], sems[parn])
            cps[par].wait()

            @plsc.parallel_loop(0, schunk, unroll=8)
            def _cbody(ci):
                v = xv[pl.ds((s * schunk + ci) * _L, _L)]
                tval = tring[par, pl.ds(ci * _L, _L)]
                d = v - tval
                chunk_fn(s * schunk + ci, v, d * d)

    def row_body(r, _):
        row = wid * rows_per + r
        if nrows % _NSUB != 0:
            row = jnp.minimum(row, nrows - 1)
        pltpu.sync_copy(x_hbm.at[row], xv)

        zero_hists()

        def p1_chunk(ci, v, d2):
            key = keys_of(v)
            bin1 = lax.shift_right_arithmetic(key, 24) + 128
            bank = (ci & (_NBANK - 1)) * (_NBIN * _L)
            addr = bank + lane_base + bin1
            plsc.addupdate_scatter(histc, [addr], ones)
            plsc.addupdate_scatter(hists, [addr], d2)
        stream_pass(row, p1_chunk)

        n_tot, s_tot1 = build_p()
        b1, p1, _pp, s1_at, _sp = find_cross(jnp.int32(n - kk))
        g8 = jnp.int32(n) - p1
        s_hi1 = s_tot1 - s1_at

        zero_hists()

        def p2_chunk(ci, v, d2):
            key = keys_of(v)
            bin1 = lax.shift_right_arithmetic(key, 24) + 128
            bin2 = lax.shift_right_arithmetic(key, 16) & 0xFF
            m = bin1 == b1
            bank = (ci & (_NBANK - 1)) * (_NBIN * _L)
            addr = bank + lane_base + bin2
            plsc.addupdate_scatter(histc, [addr], ones, mask=m)
            plsc.addupdate_scatter(hists, [addr], d2, mask=m)
        stream_pass(row, p2_chunk)

        e8, s_tot2 = build_p()
        b2, p2, pprev2, s2_at, s2_prev = find_cross(g8 + e8 - jnp.int32(kk))
        g16 = g8 + (e8 - p2)
        e16 = p2 - pprev2
        s_hi = s_hi1 + (s_tot2 - s2_at)
        s_band = s2_at - s2_prev

        vec = (jnp.where(iota == 0, s_hi, 0.0)
               + jnp.where(iota == 1, s_band, 0.0)
               + jnp.where(iota == 2, g16.astype(jnp.float32), 0.0)
               + jnp.where(iota == 3, e16.astype(jnp.float32), 0.0))
        outv[pl.ds(r * _L, _L)] = vec
        return 0

    lax.fori_loop(0, rows_per, row_body, 0)
    pltpu.sync_copy(outv, out_hbm.at[wid])


def _sc_spatial(x2d, t2d, kk):
    nrows, n = x2d.shape
    rows_per = (nrows + _NSUB - 1) // _NSUB
    mesh = plsc.VectorSubcoreMesh(core_axis_name="c", subcore_axis_name="s")
    body = functools.partial(_sc_body, nrows, n, kk, rows_per)
    f = pl.kernel(
        body,
        mesh=mesh,
        compiler_params=pltpu.CompilerParams(needs_layout_passes=False),
        out_type=jax.ShapeDtypeStruct((_NSUB, rows_per * _L), jnp.float32),
        scratch_types=[
            pltpu.VMEM((n,), jnp.float32),
            pltpu.VMEM((2, n // _NSEC), jnp.float32),
            pltpu.VMEM((_HWORDS,), jnp.int32),
            pltpu.VMEM((_HWORDS,), jnp.float32),
            pltpu.VMEM((_NBIN,), jnp.int32),
            pltpu.VMEM((_NBIN,), jnp.float32),
            pltpu.VMEM((rows_per * _L,), jnp.float32),
            pltpu.SemaphoreType.DMA,
            pltpu.SemaphoreType.DMA,
        ],
    )
    return f(x2d, t2d)


def kernel(output, target):
    B, T, C, H, W = output.shape
    hw = H * W
    kk = hw // 10
    nrows = B * T * C
    xs = output.reshape(nrows, hw)
    ts = target.reshape(nrows, hw)
    sc_part = _sc_spatial(xs, ts, kk)
    time_sse = _temporal_sse(output, target)
    rows_per = sc_part.shape[1] // _L
    p = sc_part.reshape(_NSUB * rows_per, _L)[:nrows]
    s_hi, s_band, g, e = p[:, 0], p[:, 1], p[:, 2], p[:, 3]
    spatial_sum = jnp.sum(s_hi + (kk - g) / e * s_band)
    tnorm = jnp.float32(B * C * hw * (T // _WIN))
    snorm = jnp.float32(nrows * kk)
    return time_sse / tnorm + spatial_sum / snorm


# speedup vs baseline: 1.0083x; 1.0083x over previous
import functools

import jax
import jax.numpy as jnp
from jax import lax
from jax.experimental import pallas as pl
from jax.experimental.pallas import tpu as pltpu
from jax.experimental.pallas import tpu_sc as plsc

_WIN = 4
_LANE = 128
_L = 16
_NSUB = 32
_NBIN = 256
_NSEC = 8


def _temporal_kernel(x_ref, t_ref, out_ref, acc_ref):
    step = pl.program_id(0)
    x = x_ref[0]
    tg = t_ref[0]

    @pl.when(step == 0)
    def _():
        acc_ref[0] = 0.0

    mo = jnp.maximum(jnp.maximum(x[0], x[1]), jnp.maximum(x[2], x[3]))
    mt = jnp.maximum(jnp.maximum(tg[0], tg[1]), jnp.maximum(tg[2], tg[3]))
    dt = mo - mt
    acc_ref[0] = acc_ref[0] + jnp.sum(dt * dt)

    @pl.when(step == pl.num_programs(0) - 1)
    def _():
        out_ref[0, 0] = acc_ref[0]


def _temporal_sse(output, target):
    B, T, C, H, W = output.shape
    hw = H * W
    sub = hw // _LANE
    nw = T // _WIN
    xr = output.reshape(B * nw, _WIN, C, sub, _LANE)
    tr = target.reshape(B * nw, _WIN, C, sub, _LANE)
    spec = pl.BlockSpec((1, _WIN, C, sub, _LANE),
                        lambda r: (r, 0, 0, 0, 0))
    out = pl.pallas_call(
        _temporal_kernel,
        grid=(B * nw,),
        in_specs=[spec, spec],
        out_specs=pl.BlockSpec(memory_space=pltpu.SMEM),
        out_shape=jax.ShapeDtypeStruct((1, 1), jnp.float32),
        scratch_shapes=[pltpu.SMEM((1,), jnp.float32)],
    )(xr, tr)
    return out[0, 0]


_NBANK = 1
_NCOPY = _NBANK * _L
_HWORDS = _NBIN * _NCOPY


def _sc_body(nrows, n, kk, rows_per, x_hbm, t_hbm, out_hbm,
             xv, tring, histc, hists, p_ref, ps_ref, outv, sem0, sem1):
    cid = lax.axis_index("c")
    sid = lax.axis_index("s")
    wid = sid * 2 + cid

    iota = lax.iota(jnp.int32, _L)
    lane_base = iota * _NBIN
    ones = jnp.ones((_L,), jnp.int32)
    nchunk = n // _L
    swords = n // _NSEC
    schunk = swords // _L
    i32min = jnp.int32(-2147483648)
    sems = (sem0, sem1)

    def zero_hists():
        @plsc.parallel_loop(0, _HWORDS // _L, unroll=4)
        def _z(i):
            histc[pl.ds(i * _L, _L)] = jnp.zeros((_L,), jnp.int32)
            hists[pl.ds(i * _L, _L)] = jnp.zeros((_L,), jnp.float32)

    def keys_of(v):
        bits = lax.bitcast_convert_type(v, jnp.int32)
        return jnp.where(bits < 0, bits ^ jnp.int32(0x7FFFFFFF), bits)

    def build_p():
        cum = jnp.int32(0)
        cums = jnp.float32(0.0)
        for c in range(_NBIN // _L):
            def mbody(j, acc):
                a, s = acc
                a = a + histc[pl.ds(j * _NBIN + c * _L, _L)]
                s = s + hists[pl.ds(j * _NBIN + c * _L, _L)]
                return a, s
            acc, accs = lax.fori_loop(
                0, _NCOPY, mbody,
                (jnp.zeros((_L,), jnp.int32), jnp.zeros((_L,), jnp.float32)))
            pc = plsc.cumsum(acc) + cum
            psc = plsc.cumsum(accs) + cums
            p_ref[pl.ds(c * _L, _L)] = pc
            ps_ref[pl.ds(c * _L, _L)] = psc
            cum = cum + jnp.sum(acc)
            cums = cums + jnp.sum(accs)
        return cum, cums

    def find_cross(thresh):
        found = jnp.int32(0)
        b_star = jnp.int32(0)
        p_star = jnp.int32(0)
        for c in range(_NBIN // _L):
            pc = p_ref[pl.ds(c * _L, _L)]
            m = pc > thresh
            cand = jnp.where(m, 255 - (iota + c * _L), -1)
            mx = jnp.max(cand)
            bloc = 255 - mx
            pmin = -jnp.max(jnp.where(m, -pc, i32min))
            any_m = mx >= 0
            take = (found == 0) & any_m
            b_star = jnp.where(take, bloc, b_star)
            p_star = jnp.where(take, pmin, p_star)
            found = jnp.where(any_m, jnp.int32(1), found)
        p_prev = jnp.int32(0)
        s_at = jnp.float32(0.0)
        s_prev = jnp.float32(0.0)
        for c in range(_NBIN // _L):
            bins = iota + c * _L
            pc = p_ref[pl.ds(c * _L, _L)]
            psc = ps_ref[pl.ds(c * _L, _L)]
            p_prev = p_prev + jnp.sum(jnp.where(bins == b_star - 1, pc, 0))
            s_at = s_at + jnp.sum(jnp.where(bins == b_star, psc, 0.0))
            s_prev = s_prev + jnp.sum(jnp.where(bins == b_star - 1, psc, 0.0))
        return b_star, p_star, p_prev, s_at, s_prev

    def stream_pass(row, chunk_fn):
        cps = [None, None]
        cps[0] = pltpu.async_copy(
            t_hbm.at[row, pl.ds(0, swords)], tring.at[0], sems[0])
        for s in range(_NSEC):
            par = s % 2
            if s + 1 < _NSEC:
                parn = (s + 1) % 2
                cps[parn] = pltpu.async_copy(
                    t_hbm.at[row, pl.ds((s + 1) * swords, swords)],
                    tring.at[parn], sems[parn])
            cps[par].wait()

            @plsc.parallel_loop(0, schunk, unroll=4)
            def _cbody(ci):
                v = xv[pl.ds((s * schunk + ci) * _L, _L)]
                tval = tring[par, pl.ds(ci * _L, _L)]
                d = v - tval
                chunk_fn(s * schunk + ci, v, d * d)

    def row_body(r, _):
        row = wid * rows_per + r
        if nrows % _NSUB != 0:
            row = jnp.minimum(row, nrows - 1)
        pltpu.sync_copy(x_hbm.at[row], xv)

        zero_hists()

        def p1_chunk(ci, v, d2):
            key = keys_of(v)
            bin1 = lax.shift_right_arithmetic(key, 24) + 128
            bank = (ci & (_NBANK - 1)) * (_NBIN * _L)
            addr = bank + lane_base + bin1
            plsc.addupdate_scatter(histc, [addr], ones)
            plsc.addupdate_scatter(hists, [addr], d2)
        stream_pass(row, p1_chunk)

        n_tot, s_tot1 = build_p()
        b1, p1, _pp, s1_at, _sp = find_cross(jnp.int32(n - kk))
        g8 = jnp.int32(n) - p1
        s_hi1 = s_tot1 - s1_at

        zero_hists()

        def p2_chunk(ci, v, d2):
            key = keys_of(v)
            bin1 = lax.shift_right_arithmetic(key, 24) + 128
            bin2 = lax.shift_right_arithmetic(key, 16) & 0xFF
            m = bin1 == b1
            bank = (ci & (_NBANK - 1)) * (_NBIN * _L)
            addr = bank + lane_base + bin2
            plsc.addupdate_scatter(histc, [addr], ones, mask=m)
            plsc.addupdate_scatter(hists, [addr], d2, mask=m)
        stream_pass(row, p2_chunk)

        e8, s_tot2 = build_p()
        b2, p2, pprev2, s2_at, s2_prev = find_cross(g8 + e8 - jnp.int32(kk))
        g16 = g8 + (e8 - p2)
        e16 = p2 - pprev2
        s_hi = s_hi1 + (s_tot2 - s2_at)
        s_band = s2_at - s2_prev

        vec = (jnp.where(iota == 0, s_hi, 0.0)
               + jnp.where(iota == 1, s_band, 0.0)
               + jnp.where(iota == 2, g16.astype(jnp.float32), 0.0)
               + jnp.where(iota == 3, e16.astype(jnp.float32), 0.0))
        outv[pl.ds(r * _L, _L)] = vec
        return 0

    lax.fori_loop(0, rows_per, row_body, 0)
    pltpu.sync_copy(outv, out_hbm.at[wid])


def _sc_spatial(x2d, t2d, kk):
    nrows, n = x2d.shape
    rows_per = (nrows + _NSUB - 1) // _NSUB
    mesh = plsc.VectorSubcoreMesh(core_axis_name="c", subcore_axis_name="s")
    body = functools.partial(_sc_body, nrows, n, kk, rows_per)
    f = pl.kernel(
        body,
        mesh=mesh,
        compiler_params=pltpu.CompilerParams(needs_layout_passes=False),
        out_type=jax.ShapeDtypeStruct((_NSUB, rows_per * _L), jnp.float32),
        scratch_types=[
            pltpu.VMEM((n,), jnp.float32),
            pltpu.VMEM((2, n // _NSEC), jnp.float32),
            pltpu.VMEM((_HWORDS,), jnp.int32),
            pltpu.VMEM((_HWORDS,), jnp.float32),
            pltpu.VMEM((_NBIN,), jnp.int32),
            pltpu.VMEM((_NBIN,), jnp.float32),
            pltpu.VMEM((rows_per * _L,), jnp.float32),
            pltpu.SemaphoreType.DMA,
            pltpu.SemaphoreType.DMA,
        ],
    )
    return f(x2d, t2d)


def kernel(output, target):
    B, T, C, H, W = output.shape
    hw = H * W
    kk = hw // 10
    nrows = B * T * C
    xs = output.reshape(nrows, hw)
    ts = target.reshape(nrows, hw)
    sc_part = _sc_spatial(xs, ts, kk)
    time_sse = _temporal_sse(output, target)
    rows_per = sc_part.shape[1] // _L
    p = sc_part.reshape(_NSUB * rows_per, _L)[:nrows]
    s_hi, s_band, g, e = p[:, 0], p[:, 1], p[:, 2], p[:, 3]
    spatial_sum = jnp.sum(s_hi + (kk - g) / e * s_band)
    tnorm = jnp.float32(B * C * hw * (T // _WIN))
    snorm = jnp.float32(nrows * kk)
    return time_sse / tnorm + spatial_sum / snorm


# speedup vs baseline: 1.0499x; 1.0412x over previous
import functools

import jax
import jax.numpy as jnp
from jax import lax
from jax.experimental import pallas as pl
from jax.experimental.pallas import tpu as pltpu
from jax.experimental.pallas import tpu_sc as plsc

_WIN = 4
_LANE = 128
_L = 16
_NSUB = 32
_NBIN = 256
_NSEC = 8


def _temporal_kernel(x_ref, t_ref, out_ref, acc_ref):
    step = pl.program_id(0)
    x = x_ref[0]
    tg = t_ref[0]

    @pl.when(step == 0)
    def _():
        acc_ref[0] = 0.0

    mo = jnp.maximum(jnp.maximum(x[0], x[1]), jnp.maximum(x[2], x[3]))
    mt = jnp.maximum(jnp.maximum(tg[0], tg[1]), jnp.maximum(tg[2], tg[3]))
    dt = mo - mt
    acc_ref[0] = acc_ref[0] + jnp.sum(dt * dt)

    @pl.when(step == pl.num_programs(0) - 1)
    def _():
        out_ref[0, 0] = acc_ref[0]


def _temporal_sse(output, target):
    B, T, C, H, W = output.shape
    hw = H * W
    sub = hw // _LANE
    nw = T // _WIN
    xr = output.reshape(B * nw, _WIN, C, sub, _LANE)
    tr = target.reshape(B * nw, _WIN, C, sub, _LANE)
    spec = pl.BlockSpec((1, _WIN, C, sub, _LANE),
                        lambda r: (r, 0, 0, 0, 0))
    out = pl.pallas_call(
        _temporal_kernel,
        grid=(B * nw,),
        in_specs=[spec, spec],
        out_specs=pl.BlockSpec(memory_space=pltpu.SMEM),
        out_shape=jax.ShapeDtypeStruct((1, 1), jnp.float32),
        scratch_shapes=[pltpu.SMEM((1,), jnp.float32)],
    )(xr, tr)
    return out[0, 0]


_NBANK = 1
_NCOPY = _NBANK * _L
_HWORDS = _NBIN * _NCOPY


def _sc_body(nrows, n, kk, rows_per, x_hbm, t_hbm, out_hbm,
             xv, tring, histc, hists, p_ref, ps_ref, outv,
             sem0, sem1, sem2, sem3):
    cid = lax.axis_index("c")
    sid = lax.axis_index("s")
    wid = sid * 2 + cid

    iota = lax.iota(jnp.int32, _L)
    lane_base = iota * _NBIN
    ones = jnp.ones((_L,), jnp.int32)
    nchunk = n // _L
    swords = n // _NSEC
    schunk = swords // _L
    i32min = jnp.int32(-2147483648)
    sems = (sem0, sem1)
    xsems = (sem2, sem3)

    def zero_hists():
        @plsc.parallel_loop(0, _HWORDS // _L, unroll=4)
        def _z(i):
            histc[pl.ds(i * _L, _L)] = jnp.zeros((_L,), jnp.int32)
            hists[pl.ds(i * _L, _L)] = jnp.zeros((_L,), jnp.float32)

    def keys_of(v):
        bits = lax.bitcast_convert_type(v, jnp.int32)
        return jnp.where(bits < 0, bits ^ jnp.int32(0x7FFFFFFF), bits)

    def build_p():
        cum = jnp.int32(0)
        cums = jnp.float32(0.0)
        for c in range(_NBIN // _L):
            def mbody(j, acc):
                a, s = acc
                a = a + histc[pl.ds(j * _NBIN + c * _L, _L)]
                s = s + hists[pl.ds(j * _NBIN + c * _L, _L)]
                return a, s
            acc, accs = lax.fori_loop(
                0, _NCOPY, mbody,
                (jnp.zeros((_L,), jnp.int32), jnp.zeros((_L,), jnp.float32)))
            pc = plsc.cumsum(acc) + cum
            psc = plsc.cumsum(accs) + cums
            p_ref[pl.ds(c * _L, _L)] = pc
            ps_ref[pl.ds(c * _L, _L)] = psc
            cum = cum + jnp.sum(acc)
            cums = cums + jnp.sum(accs)
        return cum, cums

    def find_cross(thresh):
        found = jnp.int32(0)
        b_star = jnp.int32(0)
        p_star = jnp.int32(0)
        for c in range(_NBIN // _L):
            pc = p_ref[pl.ds(c * _L, _L)]
            m = pc > thresh
            cand = jnp.where(m, 255 - (iota + c * _L), -1)
            mx = jnp.max(cand)
            bloc = 255 - mx
            pmin = -jnp.max(jnp.where(m, -pc, i32min))
            any_m = mx >= 0
            take = (found == 0) & any_m
            b_star = jnp.where(take, bloc, b_star)
            p_star = jnp.where(take, pmin, p_star)
            found = jnp.where(any_m, jnp.int32(1), found)
        p_prev = jnp.int32(0)
        s_at = jnp.float32(0.0)
        s_prev = jnp.float32(0.0)
        for c in range(_NBIN // _L):
            bins = iota + c * _L
            pc = p_ref[pl.ds(c * _L, _L)]
            psc = ps_ref[pl.ds(c * _L, _L)]
            p_prev = p_prev + jnp.sum(jnp.where(bins == b_star - 1, pc, 0))
            s_at = s_at + jnp.sum(jnp.where(bins == b_star, psc, 0.0))
            s_prev = s_prev + jnp.sum(jnp.where(bins == b_star - 1, psc, 0.0))
        return b_star, p_star, p_prev, s_at, s_prev

    def stream_pass(row, chunk_fn, load_x=False):
        cps = [None, None]
        xcps = [None, None]
        cps[0] = pltpu.async_copy(
            t_hbm.at[row, pl.ds(0, swords)], tring.at[0], sems[0])
        if load_x:
            xcps[0] = pltpu.async_copy(
                x_hbm.at[row, pl.ds(0, swords)],
                xv.at[pl.ds(0, swords)], xsems[0])
        for s in range(_NSEC):
            par = s % 2
            if s + 1 < _NSEC:
                parn = (s + 1) % 2
                cps[parn] = pltpu.async_copy(
                    t_hbm.at[row, pl.ds((s + 1) * swords, swords)],
                    tring.at[parn], sems[parn])
                if load_x:
                    xcps[parn] = pltpu.async_copy(
                        x_hbm.at[row, pl.ds((s + 1) * swords, swords)],
                        xv.at[pl.ds((s + 1) * swords, swords)], xsems[parn])
            cps[par].wait()
            if load_x:
                xcps[par].wait()

            @plsc.parallel_loop(0, schunk, unroll=4)
            def _cbody(ci):
                v = xv[pl.ds((s * schunk + ci) * _L, _L)]
                tval = tring[par, pl.ds(ci * _L, _L)]
                d = v - tval
                chunk_fn(s * schunk + ci, v, d * d)

    def row_body(r, _):
        row = wid * rows_per + r
        if nrows % _NSUB != 0:
            row = jnp.minimum(row, nrows - 1)
        zero_hists()

        def p1_chunk(ci, v, d2):
            key = keys_of(v)
            bin1 = lax.shift_right_arithmetic(key, 24) + 128
            bank = (ci & (_NBANK - 1)) * (_NBIN * _L)
            addr = bank + lane_base + bin1
            plsc.addupdate_scatter(histc, [addr], ones)
            plsc.addupdate_scatter(hists, [addr], d2)
        stream_pass(row, p1_chunk, load_x=True)

        n_tot, s_tot1 = build_p()
        b1, p1, _pp, s1_at, _sp = find_cross(jnp.int32(n - kk))
        g8 = jnp.int32(n) - p1
        s_hi1 = s_tot1 - s1_at

        zero_hists()

        def p2_chunk(ci, v, d2):
            key = keys_of(v)
            bin1 = lax.shift_right_arithmetic(key, 24) + 128
            bin2 = lax.shift_right_arithmetic(key, 16) & 0xFF
            m = bin1 == b1
            bank = (ci & (_NBANK - 1)) * (_NBIN * _L)
            addr = bank + lane_base + bin2
            plsc.addupdate_scatter(histc, [addr], ones, mask=m)
            plsc.addupdate_scatter(hists, [addr], d2, mask=m)
        stream_pass(row, p2_chunk)

        e8, s_tot2 = build_p()
        b2, p2, pprev2, s2_at, s2_prev = find_cross(g8 + e8 - jnp.int32(kk))
        g16 = g8 + (e8 - p2)
        e16 = p2 - pprev2
        s_hi = s_hi1 + (s_tot2 - s2_at)
        s_band = s2_at - s2_prev

        vec = (jnp.where(iota == 0, s_hi, 0.0)
               + jnp.where(iota == 1, s_band, 0.0)
               + jnp.where(iota == 2, g16.astype(jnp.float32), 0.0)
               + jnp.where(iota == 3, e16.astype(jnp.float32), 0.0))
        outv[pl.ds(r * _L, _L)] = vec
        return 0

    lax.fori_loop(0, rows_per, row_body, 0)
    pltpu.sync_copy(outv, out_hbm.at[wid])


def _sc_spatial(x2d, t2d, kk):
    nrows, n = x2d.shape
    rows_per = (nrows + _NSUB - 1) // _NSUB
    mesh = plsc.VectorSubcoreMesh(core_axis_name="c", subcore_axis_name="s")
    body = functools.partial(_sc_body, nrows, n, kk, rows_per)
    f = pl.kernel(
        body,
        mesh=mesh,
        compiler_params=pltpu.CompilerParams(needs_layout_passes=False),
        out_type=jax.ShapeDtypeStruct((_NSUB, rows_per * _L), jnp.float32),
        scratch_types=[
            pltpu.VMEM((n,), jnp.float32),
            pltpu.VMEM((2, n // _NSEC), jnp.float32),
            pltpu.VMEM((_HWORDS,), jnp.int32),
            pltpu.VMEM((_HWORDS,), jnp.float32),
            pltpu.VMEM((_NBIN,), jnp.int32),
            pltpu.VMEM((_NBIN,), jnp.float32),
            pltpu.VMEM((rows_per * _L,), jnp.float32),
            pltpu.SemaphoreType.DMA,
            pltpu.SemaphoreType.DMA,
            pltpu.SemaphoreType.DMA,
            pltpu.SemaphoreType.DMA,
        ],
    )
    return f(x2d, t2d)


def kernel(output, target):
    B, T, C, H, W = output.shape
    hw = H * W
    kk = hw // 10
    nrows = B * T * C
    xs = output.reshape(nrows, hw)
    ts = target.reshape(nrows, hw)
    sc_part = _sc_spatial(xs, ts, kk)
    time_sse = _temporal_sse(output, target)
    rows_per = sc_part.shape[1] // _L
    p = sc_part.reshape(_NSUB * rows_per, _L)[:nrows]
    s_hi, s_band, g, e = p[:, 0], p[:, 1], p[:, 2], p[:, 3]
    spatial_sum = jnp.sum(s_hi + (kk - g) / e * s_band)
    tnorm = jnp.float32(B * C * hw * (T // _WIN))
    snorm = jnp.float32(nrows * kk)
    return time_sse / tnorm + spatial_sum / snorm


# speedup vs baseline: 1.1696x; 1.1140x over previous
import functools

import jax
import jax.numpy as jnp
from jax import lax
from jax.experimental import pallas as pl
from jax.experimental.pallas import tpu as pltpu
from jax.experimental.pallas import tpu_sc as plsc

_WIN = 4
_LANE = 128
_L = 16
_NSUB = 32
_NBIN = 256
_NSEC = 8


def _tc_kernel(nsplit, kk, x_ref, t_ref, out_ref, acc_ref):
    step = pl.program_id(0)
    x = x_ref[0]
    tg = t_ref[0]
    nc = x.shape[1]

    @pl.when(step == 0)
    def _():
        acc_ref[0] = 0.0
        acc_ref[1] = 0.0

    mo = jnp.maximum(jnp.maximum(x[0], x[1]), jnp.maximum(x[2], x[3]))
    mt = jnp.maximum(jnp.maximum(tg[0], tg[1]), jnp.maximum(tg[2], tg[3]))
    dt = mo - mt
    acc_ref[0] = acc_ref[0] + jnp.sum(dt * dt)

    @pl.when(step < nsplit)
    def _():
        bits = lax.bitcast_convert_type(x, jnp.int32)
        key = jnp.where(bits < 0, bits ^ jnp.int32(0x7FFFFFFF), bits)
        key16 = lax.shift_right_arithmetic(key, 16)

        def body(i, lohi):
            lo, hi = lohi
            mid = lax.shift_right_arithmetic(lo + hi, 1)
            cnt = jnp.sum((key16 > mid).astype(jnp.int32), axis=(2, 3),
                          keepdims=True)
            pred = cnt < kk
            return jnp.where(pred, lo, mid), jnp.where(pred, mid, hi)

        lo0 = jnp.full((_WIN, nc, 1, 1), -32769, jnp.int32)
        hi0 = jnp.full((_WIN, nc, 1, 1), 32767, jnp.int32)
        _, hi = lax.fori_loop(0, 16, body, (lo0, hi0))

        d2 = (x - tg) * (x - tg)
        mhi = key16 > hi
        mband = key16 == hi
        s_hi = jnp.sum(jnp.where(mhi, d2, 0.0))
        s_band = jnp.sum(jnp.where(mband, d2, 0.0), axis=(2, 3),
                         keepdims=True)
        g = jnp.sum(mhi.astype(jnp.float32), axis=(2, 3), keepdims=True)
        e = jnp.sum(mband.astype(jnp.float32), axis=(2, 3), keepdims=True)
        w = (jnp.float32(kk) - g) / e
        acc_ref[1] = acc_ref[1] + s_hi + jnp.sum(w * s_band)

    @pl.when(step == pl.num_programs(0) - 1)
    def _():
        out_ref[0, 0] = acc_ref[0]
        out_ref[0, 1] = acc_ref[1]


def _tc_part(output, target, nsplit, kk):
    B, T, C, H, W = output.shape
    hw = H * W
    sub = hw // _LANE
    nw = T // _WIN
    xr = output.reshape(B * nw, _WIN, C, sub, _LANE)
    tr = target.reshape(B * nw, _WIN, C, sub, _LANE)
    spec = pl.BlockSpec((1, _WIN, C, sub, _LANE),
                        lambda r: (r, 0, 0, 0, 0))
    out = pl.pallas_call(
        functools.partial(_tc_kernel, nsplit, kk),
        grid=(B * nw,),
        in_specs=[spec, spec],
        out_specs=pl.BlockSpec(memory_space=pltpu.SMEM),
        out_shape=jax.ShapeDtypeStruct((1, 2), jnp.float32),
        scratch_shapes=[pltpu.SMEM((2,), jnp.float32)],
    )(xr, tr)
    return out[0, 0], out[0, 1]


_NBANK = 1
_NCOPY = _NBANK * _L
_HWORDS = _NBIN * _NCOPY


def _sc_body(nsc, roff, n, kk, rows_per, x_hbm, t_hbm, out_hbm,
             xv, tring, histc, hists, p_ref, ps_ref, outv,
             sem0, sem1, sem2, sem3):
    cid = lax.axis_index("c")
    sid = lax.axis_index("s")
    wid = sid * 2 + cid

    iota = lax.iota(jnp.int32, _L)
    lane_base = iota * _NBIN
    ones = jnp.ones((_L,), jnp.int32)
    nchunk = n // _L
    swords = n // _NSEC
    schunk = swords // _L
    i32min = jnp.int32(-2147483648)
    sems = (sem0, sem1)
    xsems = (sem2, sem3)

    def zero_hists():
        @plsc.parallel_loop(0, _HWORDS // _L, unroll=4)
        def _z(i):
            histc[pl.ds(i * _L, _L)] = jnp.zeros((_L,), jnp.int32)
            hists[pl.ds(i * _L, _L)] = jnp.zeros((_L,), jnp.float32)

    def keys_of(v):
        bits = lax.bitcast_convert_type(v, jnp.int32)
        return jnp.where(bits < 0, bits ^ jnp.int32(0x7FFFFFFF), bits)

    def build_p():
        cum = jnp.int32(0)
        cums = jnp.float32(0.0)
        for c in range(_NBIN // _L):
            def mbody(j, acc):
                a, s = acc
                a = a + histc[pl.ds(j * _NBIN + c * _L, _L)]
                s = s + hists[pl.ds(j * _NBIN + c * _L, _L)]
                return a, s
            acc, accs = lax.fori_loop(
                0, _NCOPY, mbody,
                (jnp.zeros((_L,), jnp.int32), jnp.zeros((_L,), jnp.float32)))
            pc = plsc.cumsum(acc) + cum
            psc = plsc.cumsum(accs) + cums
            p_ref[pl.ds(c * _L, _L)] = pc
            ps_ref[pl.ds(c * _L, _L)] = psc
            cum = cum + jnp.sum(acc)
            cums = cums + jnp.sum(accs)
        return cum, cums

    def find_cross(thresh):
        found = jnp.int32(0)
        b_star = jnp.int32(0)
        p_star = jnp.int32(0)
        for c in range(_NBIN // _L):
            pc = p_ref[pl.ds(c * _L, _L)]
            m = pc > thresh
            cand = jnp.where(m, 255 - (iota + c * _L), -1)
            mx = jnp.max(cand)
            bloc = 255 - mx
            pmin = -jnp.max(jnp.where(m, -pc, i32min))
            any_m = mx >= 0
            take = (found == 0) & any_m
            b_star = jnp.where(take, bloc, b_star)
            p_star = jnp.where(take, pmin, p_star)
            found = jnp.where(any_m, jnp.int32(1), found)
        p_prev = jnp.int32(0)
        s_at = jnp.float32(0.0)
        s_prev = jnp.float32(0.0)
        for c in range(_NBIN // _L):
            bins = iota + c * _L
            pc = p_ref[pl.ds(c * _L, _L)]
            psc = ps_ref[pl.ds(c * _L, _L)]
            p_prev = p_prev + jnp.sum(jnp.where(bins == b_star - 1, pc, 0))
            s_at = s_at + jnp.sum(jnp.where(bins == b_star, psc, 0.0))
            s_prev = s_prev + jnp.sum(jnp.where(bins == b_star - 1, psc, 0.0))
        return b_star, p_star, p_prev, s_at, s_prev

    def stream_pass(row, chunk_fn, load_x=False):
        cps = [None, None]
        xcps = [None, None]
        cps[0] = pltpu.async_copy(
            t_hbm.at[row, pl.ds(0, swords)], tring.at[0], sems[0])
        if load_x:
            xcps[0] = pltpu.async_copy(
                x_hbm.at[row, pl.ds(0, swords)],
                xv.at[pl.ds(0, swords)], xsems[0])
        for s in range(_NSEC):
            par = s % 2
            if s + 1 < _NSEC:
                parn = (s + 1) % 2
                cps[parn] = pltpu.async_copy(
                    t_hbm.at[row, pl.ds((s + 1) * swords, swords)],
                    tring.at[parn], sems[parn])
                if load_x:
                    xcps[parn] = pltpu.async_copy(
                        x_hbm.at[row, pl.ds((s + 1) * swords, swords)],
                        xv.at[pl.ds((s + 1) * swords, swords)], xsems[parn])
            cps[par].wait()
            if load_x:
                xcps[par].wait()

            @plsc.parallel_loop(0, schunk, unroll=4)
            def _cbody(ci):
                v = xv[pl.ds((s * schunk + ci) * _L, _L)]
                tval = tring[par, pl.ds(ci * _L, _L)]
                d = v - tval
                chunk_fn(s * schunk + ci, v, d * d)

    def row_body(r, _):
        row = roff + jnp.minimum(wid * rows_per + r, nsc - 1)
        zero_hists()

        def p1_chunk(ci, v, d2):
            key = keys_of(v)
            bin1 = lax.shift_right_arithmetic(key, 24) + 128
            bank = (ci & (_NBANK - 1)) * (_NBIN * _L)
            addr = bank + lane_base + bin1
            plsc.addupdate_scatter(histc, [addr], ones)
            plsc.addupdate_scatter(hists, [addr], d2)
        stream_pass(row, p1_chunk, load_x=True)

        n_tot, s_tot1 = build_p()
        b1, p1, _pp, s1_at, _sp = find_cross(jnp.int32(n - kk))
        g8 = jnp.int32(n) - p1
        s_hi1 = s_tot1 - s1_at

        zero_hists()

        def p2_chunk(ci, v, d2):
            key = keys_of(v)
            bin1 = lax.shift_right_arithmetic(key, 24) + 128
            bin2 = lax.shift_right_arithmetic(key, 16) & 0xFF
            m = bin1 == b1
            bank = (ci & (_NBANK - 1)) * (_NBIN * _L)
            addr = bank + lane_base + bin2
            plsc.addupdate_scatter(histc, [addr], ones, mask=m)
            plsc.addupdate_scatter(hists, [addr], d2, mask=m)
        stream_pass(row, p2_chunk)

        e8, s_tot2 = build_p()
        b2, p2, pprev2, s2_at, s2_prev = find_cross(g8 + e8 - jnp.int32(kk))
        g16 = g8 + (e8 - p2)
        e16 = p2 - pprev2
        s_hi = s_hi1 + (s_tot2 - s2_at)
        s_band = s2_at - s2_prev

        vec = (jnp.where(iota == 0, s_hi, 0.0)
               + jnp.where(iota == 1, s_band, 0.0)
               + jnp.where(iota == 2, g16.astype(jnp.float32), 0.0)
               + jnp.where(iota == 3, e16.astype(jnp.float32), 0.0))
        outv[pl.ds(r * _L, _L)] = vec
        return 0

    lax.fori_loop(0, rows_per, row_body, 0)
    pltpu.sync_copy(outv, out_hbm.at[wid])


def _sc_spatial(x2d, t2d, kk, roff):
    nrows, n = x2d.shape
    nsc = nrows - roff
    rows_per = (nsc + _NSUB - 1) // _NSUB
    mesh = plsc.VectorSubcoreMesh(core_axis_name="c", subcore_axis_name="s")
    body = functools.partial(_sc_body, nsc, roff, n, kk, rows_per)
    f = pl.kernel(
        body,
        mesh=mesh,
        compiler_params=pltpu.CompilerParams(needs_layout_passes=False),
        out_type=jax.ShapeDtypeStruct((_NSUB, rows_per * _L), jnp.float32),
        scratch_types=[
            pltpu.VMEM((n,), jnp.float32),
            pltpu.VMEM((2, n // _NSEC), jnp.float32),
            pltpu.VMEM((_HWORDS,), jnp.int32),
            pltpu.VMEM((_HWORDS,), jnp.float32),
            pltpu.VMEM((_NBIN,), jnp.int32),
            pltpu.VMEM((_NBIN,), jnp.float32),
            pltpu.VMEM((rows_per * _L,), jnp.float32),
            pltpu.SemaphoreType.DMA,
            pltpu.SemaphoreType.DMA,
            pltpu.SemaphoreType.DMA,
            pltpu.SemaphoreType.DMA,
        ],
    )
    return f(x2d, t2d)


def kernel(output, target):
    B, T, C, H, W = output.shape
    hw = H * W
    kk = hw // 10
    nrows = B * T * C
    nblocks = B * (T // _WIN)
    nsplit = (nblocks * 11) // 16
    roff = nsplit * _WIN * C
    xs = output.reshape(nrows, hw)
    ts = target.reshape(nrows, hw)
    sc_part = _sc_spatial(xs, ts, kk, roff)
    time_sse, tc_spatial = _tc_part(output, target, nsplit, kk)
    rows_per = sc_part.shape[1] // _L
    p = sc_part.reshape(_NSUB * rows_per, _L)[:nrows - roff]
    s_hi, s_band, g, e = p[:, 0], p[:, 1], p[:, 2], p[:, 3]
    spatial_sum = tc_spatial + jnp.sum(s_hi + (kk - g) / e * s_band)
    tnorm = jnp.float32(B * C * hw * (T // _WIN))
    snorm = jnp.float32(nrows * kk)
    return time_sse / tnorm + spatial_sum / snorm


# speedup vs baseline: 1.2491x; 1.0680x over previous
import functools

import jax
import jax.numpy as jnp
from jax import lax
from jax.experimental import pallas as pl
from jax.experimental.pallas import tpu as pltpu
from jax.experimental.pallas import tpu_sc as plsc

_WIN = 4
_LANE = 128
_L = 16
_NSUB = 32
_NBIN = 256
_NSEC = 8


def _tc_kernel(nsplit, kk, x_ref, t_ref, out_ref, acc_ref):
    step = pl.program_id(0)
    x = x_ref[0]
    tg = t_ref[0]
    nc = x.shape[1]

    @pl.when(step == 0)
    def _():
        acc_ref[0] = 0.0
        acc_ref[1] = 0.0

    mo = jnp.maximum(jnp.maximum(x[0], x[1]), jnp.maximum(x[2], x[3]))
    mt = jnp.maximum(jnp.maximum(tg[0], tg[1]), jnp.maximum(tg[2], tg[3]))
    dt = mo - mt
    acc_ref[0] = acc_ref[0] + jnp.sum(dt * dt)

    @pl.when(step < nsplit)
    def _():
        bits = lax.bitcast_convert_type(x, jnp.int32)
        key = jnp.where(bits < 0, bits ^ jnp.int32(0x7FFFFFFF), bits)
        key16 = lax.shift_right_arithmetic(key, 16)

        def body(i, lohi):
            lo, hi = lohi
            mid = lax.shift_right_arithmetic(lo + hi, 1)
            cnt = jnp.sum((key16 > mid).astype(jnp.int32), axis=(2, 3),
                          keepdims=True)
            pred = cnt < kk
            return jnp.where(pred, lo, mid), jnp.where(pred, mid, hi)

        lo0 = jnp.full((_WIN, nc, 1, 1), -32769, jnp.int32)
        hi0 = jnp.full((_WIN, nc, 1, 1), 32767, jnp.int32)
        _, hi = lax.fori_loop(0, 16, body, (lo0, hi0))

        d2 = (x - tg) * (x - tg)
        mhi = key16 > hi
        mband = key16 == hi
        s_hi = jnp.sum(jnp.where(mhi, d2, 0.0))
        s_band = jnp.sum(jnp.where(mband, d2, 0.0), axis=(2, 3),
                         keepdims=True)
        g = jnp.sum(mhi.astype(jnp.float32), axis=(2, 3), keepdims=True)
        e = jnp.sum(mband.astype(jnp.float32), axis=(2, 3), keepdims=True)
        w = (jnp.float32(kk) - g) / e
        acc_ref[1] = acc_ref[1] + s_hi + jnp.sum(w * s_band)

    @pl.when(step == pl.num_programs(0) - 1)
    def _():
        out_ref[0, 0] = acc_ref[0]
        out_ref[0, 1] = acc_ref[1]


def _tc_part(output, target, nsplit, kk):
    B, T, C, H, W = output.shape
    hw = H * W
    sub = hw // _LANE
    nw = T // _WIN
    xr = output.reshape(B * nw, _WIN, C, sub, _LANE)
    tr = target.reshape(B * nw, _WIN, C, sub, _LANE)
    spec = pl.BlockSpec((1, _WIN, C, sub, _LANE),
                        lambda r: (r, 0, 0, 0, 0))
    out = pl.pallas_call(
        functools.partial(_tc_kernel, nsplit, kk),
        grid=(B * nw,),
        in_specs=[spec, spec],
        out_specs=pl.BlockSpec(memory_space=pltpu.SMEM),
        out_shape=jax.ShapeDtypeStruct((1, 2), jnp.float32),
        scratch_shapes=[pltpu.SMEM((2,), jnp.float32)],
    )(xr, tr)
    return out[0, 0], out[0, 1]


_NBANK = 1
_NCOPY = _NBANK * _L
_HWORDS = _NBIN * _NCOPY


def _sc_body(nsc, roff, n, kk, rows_per, x_hbm, t_hbm, out_hbm,
             xv, tring, histc, hists, p_ref, ps_ref, outv,
             sem0, sem1, sem2, sem3):
    cid = lax.axis_index("c")
    sid = lax.axis_index("s")
    wid = sid * 2 + cid

    iota = lax.iota(jnp.int32, _L)
    lane_base = iota * _NBIN
    ones = jnp.ones((_L,), jnp.int32)
    nchunk = n // _L
    swords = n // _NSEC
    schunk = swords // _L
    i32min = jnp.int32(-2147483648)
    sems = (sem0, sem1)
    xsems = (sem2, sem3)

    def zero_hists():
        @plsc.parallel_loop(0, _HWORDS // _L, unroll=4)
        def _z(i):
            histc[pl.ds(i * _L, _L)] = jnp.zeros((_L,), jnp.int32)
            hists[pl.ds(i * _L, _L)] = jnp.zeros((_L,), jnp.float32)

    def keys_of(v):
        bits = lax.bitcast_convert_type(v, jnp.int32)
        return jnp.where(bits < 0, bits ^ jnp.int32(0x7FFFFFFF), bits)

    def build_p():
        cum = jnp.int32(0)
        cums = jnp.float32(0.0)
        for c in range(_NBIN // _L):
            def mbody(j, acc):
                a, s = acc
                a = a + histc[pl.ds(j * _NBIN + c * _L, _L)]
                s = s + hists[pl.ds(j * _NBIN + c * _L, _L)]
                return a, s
            acc, accs = lax.fori_loop(
                0, _NCOPY, mbody,
                (jnp.zeros((_L,), jnp.int32), jnp.zeros((_L,), jnp.float32)))
            pc = plsc.cumsum(acc) + cum
            psc = plsc.cumsum(accs) + cums
            p_ref[pl.ds(c * _L, _L)] = pc
            ps_ref[pl.ds(c * _L, _L)] = psc
            cum = cum + jnp.sum(acc)
            cums = cums + jnp.sum(accs)
        return cum, cums

    def find_cross(thresh):
        found = jnp.int32(0)
        b_star = jnp.int32(0)
        p_star = jnp.int32(0)
        for c in range(_NBIN // _L):
            pc = p_ref[pl.ds(c * _L, _L)]
            m = pc > thresh
            cand = jnp.where(m, 255 - (iota + c * _L), -1)
            mx = jnp.max(cand)
            bloc = 255 - mx
            pmin = -jnp.max(jnp.where(m, -pc, i32min))
            any_m = mx >= 0
            take = (found == 0) & any_m
            b_star = jnp.where(take, bloc, b_star)
            p_star = jnp.where(take, pmin, p_star)
            found = jnp.where(any_m, jnp.int32(1), found)
        p_prev = jnp.int32(0)
        s_at = jnp.float32(0.0)
        s_prev = jnp.float32(0.0)
        for c in range(_NBIN // _L):
            bins = iota + c * _L
            pc = p_ref[pl.ds(c * _L, _L)]
            psc = ps_ref[pl.ds(c * _L, _L)]
            p_prev = p_prev + jnp.sum(jnp.where(bins == b_star - 1, pc, 0))
            s_at = s_at + jnp.sum(jnp.where(bins == b_star, psc, 0.0))
            s_prev = s_prev + jnp.sum(jnp.where(bins == b_star - 1, psc, 0.0))
        return b_star, p_star, p_prev, s_at, s_prev

    def stream_pass(row, chunk_fn, load_x=False):
        cps = [None, None]
        xcps = [None, None]
        cps[0] = pltpu.async_copy(
            t_hbm.at[row, pl.ds(0, swords)], tring.at[0], sems[0])
        if load_x:
            xcps[0] = pltpu.async_copy(
                x_hbm.at[row, pl.ds(0, swords)],
                xv.at[pl.ds(0, swords)], xsems[0])
        for s in range(_NSEC):
            par = s % 2
            if s + 1 < _NSEC:
                parn = (s + 1) % 2
                cps[parn] = pltpu.async_copy(
                    t_hbm.at[row, pl.ds((s + 1) * swords, swords)],
                    tring.at[parn], sems[parn])
                if load_x:
                    xcps[parn] = pltpu.async_copy(
                        x_hbm.at[row, pl.ds((s + 1) * swords, swords)],
                        xv.at[pl.ds((s + 1) * swords, swords)], xsems[parn])
            cps[par].wait()
            if load_x:
                xcps[par].wait()

            @plsc.parallel_loop(0, schunk, unroll=4)
            def _cbody(ci):
                v = xv[pl.ds((s * schunk + ci) * _L, _L)]
                tval = tring[par, pl.ds(ci * _L, _L)]
                d = v - tval
                chunk_fn(s * schunk + ci, v, d * d)

    def row_body(r, _):
        row = roff + jnp.minimum(wid * rows_per + r, nsc - 1)
        zero_hists()

        def p1_chunk(ci, v, d2):
            key = keys_of(v)
            bin1 = lax.shift_right_arithmetic(key, 24) + 128
            bank = (ci & (_NBANK - 1)) * (_NBIN * _L)
            addr = bank + lane_base + bin1
            plsc.addupdate_scatter(histc, [addr], ones)
            plsc.addupdate_scatter(hists, [addr], d2)
        stream_pass(row, p1_chunk, load_x=True)

        n_tot, s_tot1 = build_p()
        b1, p1, _pp, s1_at, _sp = find_cross(jnp.int32(n - kk))
        g8 = jnp.int32(n) - p1
        s_hi1 = s_tot1 - s1_at

        zero_hists()

        def p2_chunk(ci, v, d2):
            key = keys_of(v)
            bin1 = lax.shift_right_arithmetic(key, 24) + 128
            bin2 = lax.shift_right_arithmetic(key, 16) & 0xFF
            m = bin1 == b1
            bank = (ci & (_NBANK - 1)) * (_NBIN * _L)
            addr = bank + lane_base + bin2
            plsc.addupdate_scatter(histc, [addr], ones, mask=m)
            plsc.addupdate_scatter(hists, [addr], d2, mask=m)
        stream_pass(row, p2_chunk)

        e8, s_tot2 = build_p()
        b2, p2, pprev2, s2_at, s2_prev = find_cross(g8 + e8 - jnp.int32(kk))
        g16 = g8 + (e8 - p2)
        e16 = p2 - pprev2
        s_hi = s_hi1 + (s_tot2 - s2_at)
        s_band = s2_at - s2_prev

        vec = (jnp.where(iota == 0, s_hi, 0.0)
               + jnp.where(iota == 1, s_band, 0.0)
               + jnp.where(iota == 2, g16.astype(jnp.float32), 0.0)
               + jnp.where(iota == 3, e16.astype(jnp.float32), 0.0))
        outv[pl.ds(r * _L, _L)] = vec
        return 0

    lax.fori_loop(0, rows_per, row_body, 0)
    pltpu.sync_copy(outv, out_hbm.at[wid])


def _sc_spatial(x2d, t2d, kk, roff):
    nrows, n = x2d.shape
    nsc = nrows - roff
    rows_per = (nsc + _NSUB - 1) // _NSUB
    mesh = plsc.VectorSubcoreMesh(core_axis_name="c", subcore_axis_name="s")
    body = functools.partial(_sc_body, nsc, roff, n, kk, rows_per)
    f = pl.kernel(
        body,
        mesh=mesh,
        compiler_params=pltpu.CompilerParams(needs_layout_passes=False),
        out_type=jax.ShapeDtypeStruct((_NSUB, rows_per * _L), jnp.float32),
        scratch_types=[
            pltpu.VMEM((n,), jnp.float32),
            pltpu.VMEM((2, n // _NSEC), jnp.float32),
            pltpu.VMEM((_HWORDS,), jnp.int32),
            pltpu.VMEM((_HWORDS,), jnp.float32),
            pltpu.VMEM((_NBIN,), jnp.int32),
            pltpu.VMEM((_NBIN,), jnp.float32),
            pltpu.VMEM((rows_per * _L,), jnp.float32),
            pltpu.SemaphoreType.DMA,
            pltpu.SemaphoreType.DMA,
            pltpu.SemaphoreType.DMA,
            pltpu.SemaphoreType.DMA,
        ],
    )
    return f(x2d, t2d)


def kernel(output, target):
    B, T, C, H, W = output.shape
    hw = H * W
    kk = hw // 10
    nrows = B * T * C
    nblocks = B * (T // _WIN)
    nsplit = (nblocks * 8) // 16
    roff = nsplit * _WIN * C
    xs = output.reshape(nrows, hw)
    ts = target.reshape(nrows, hw)
    sc_part = _sc_spatial(xs, ts, kk, roff)
    time_sse, tc_spatial = _tc_part(output, target, nsplit, kk)
    rows_per = sc_part.shape[1] // _L
    p = sc_part.reshape(_NSUB * rows_per, _L)[:nrows - roff]
    s_hi, s_band, g, e = p[:, 0], p[:, 1], p[:, 2], p[:, 3]
    spatial_sum = tc_spatial + jnp.sum(s_hi + (kk - g) / e * s_band)
    tnorm = jnp.float32(B * C * hw * (T // _WIN))
    snorm = jnp.float32(nrows * kk)
    return time_sse / tnorm + spatial_sum / snorm


# speedup vs baseline: 1.2747x; 1.0205x over previous
import functools

import jax
import jax.numpy as jnp
from jax import lax
from jax.experimental import pallas as pl
from jax.experimental.pallas import tpu as pltpu
from jax.experimental.pallas import tpu_sc as plsc

_WIN = 4
_LANE = 128
_L = 16
_NSUB = 32
_NBIN = 256
_NSEC = 8


def _tc_kernel(nsplit, kk, x_ref, t_ref, out_ref, acc_ref):
    step = pl.program_id(0)
    x = x_ref[0]
    tg = t_ref[0]
    nc = x.shape[1]

    @pl.when(step == 0)
    def _():
        acc_ref[0] = 0.0
        acc_ref[1] = 0.0

    mo = jnp.maximum(jnp.maximum(x[0], x[1]), jnp.maximum(x[2], x[3]))
    mt = jnp.maximum(jnp.maximum(tg[0], tg[1]), jnp.maximum(tg[2], tg[3]))
    dt = mo - mt
    acc_ref[0] = acc_ref[0] + jnp.sum(dt * dt)

    @pl.when(step < nsplit)
    def _():
        bits = lax.bitcast_convert_type(x, jnp.int32)
        key = jnp.where(bits < 0, bits ^ jnp.int32(0x7FFFFFFF), bits)
        key16 = lax.shift_right_arithmetic(key, 16)

        def body(i, lohi):
            lo, hi = lohi
            mid = lax.shift_right_arithmetic(lo + hi, 1)
            cnt = jnp.sum((key16 > mid).astype(jnp.int32), axis=(2, 3),
                          keepdims=True)
            pred = cnt < kk
            return jnp.where(pred, lo, mid), jnp.where(pred, mid, hi)

        lo0 = jnp.full((_WIN, nc, 1, 1), -32769, jnp.int32)
        hi0 = jnp.full((_WIN, nc, 1, 1), 32767, jnp.int32)
        _, hi = lax.fori_loop(0, 16, body, (lo0, hi0))

        d2 = (x - tg) * (x - tg)
        mhi = key16 > hi
        mband = key16 == hi
        s_hi = jnp.sum(jnp.where(mhi, d2, 0.0))
        s_band = jnp.sum(jnp.where(mband, d2, 0.0), axis=(2, 3),
                         keepdims=True)
        g = jnp.sum(mhi.astype(jnp.float32), axis=(2, 3), keepdims=True)
        e = jnp.sum(mband.astype(jnp.float32), axis=(2, 3), keepdims=True)
        w = (jnp.float32(kk) - g) / e
        acc_ref[1] = acc_ref[1] + s_hi + jnp.sum(w * s_band)

    @pl.when(step == pl.num_programs(0) - 1)
    def _():
        out_ref[0, 0] = acc_ref[0]
        out_ref[0, 1] = acc_ref[1]


def _tc_part(output, target, nsplit, kk):
    B, T, C, H, W = output.shape
    hw = H * W
    sub = hw // _LANE
    nw = T // _WIN
    xr = output.reshape(B * nw, _WIN, C, sub, _LANE)
    tr = target.reshape(B * nw, _WIN, C, sub, _LANE)
    spec = pl.BlockSpec((1, _WIN, C, sub, _LANE),
                        lambda r: (r, 0, 0, 0, 0))
    out = pl.pallas_call(
        functools.partial(_tc_kernel, nsplit, kk),
        grid=(B * nw,),
        in_specs=[spec, spec],
        out_specs=pl.BlockSpec(memory_space=pltpu.SMEM),
        out_shape=jax.ShapeDtypeStruct((1, 2), jnp.float32),
        scratch_shapes=[pltpu.SMEM((2,), jnp.float32)],
    )(xr, tr)
    return out[0, 0], out[0, 1]


_NBANK = 1
_NCOPY = _NBANK * _L
_HWORDS = _NBIN * _NCOPY


def _sc_body(nsc, roff, n, kk, rows_per, x_hbm, t_hbm, out_hbm,
             xv, tring, histc, hists, p_ref, ps_ref, outv,
             sem0, sem1, sem2, sem3):
    cid = lax.axis_index("c")
    sid = lax.axis_index("s")
    wid = sid * 2 + cid

    iota = lax.iota(jnp.int32, _L)
    lane_base = iota * _NBIN
    ones = jnp.ones((_L,), jnp.int32)
    nchunk = n // _L
    swords = n // _NSEC
    schunk = swords // _L
    i32min = jnp.int32(-2147483648)
    sems = (sem0, sem1)
    xsems = (sem2, sem3)

    def zero_hists():
        @plsc.parallel_loop(0, _HWORDS // _L, unroll=4)
        def _z(i):
            histc[pl.ds(i * _L, _L)] = jnp.zeros((_L,), jnp.int32)
            hists[pl.ds(i * _L, _L)] = jnp.zeros((_L,), jnp.float32)

    def keys_of(v):
        bits = lax.bitcast_convert_type(v, jnp.int32)
        return jnp.where(bits < 0, bits ^ jnp.int32(0x7FFFFFFF), bits)

    def build_p():
        cum = jnp.int32(0)
        cums = jnp.float32(0.0)
        for c in range(_NBIN // _L):
            def mbody(j, acc):
                a, s = acc
                a = a + histc[pl.ds(j * _NBIN + c * _L, _L)]
                s = s + hists[pl.ds(j * _NBIN + c * _L, _L)]
                return a, s
            acc, accs = lax.fori_loop(
                0, _NCOPY, mbody,
                (jnp.zeros((_L,), jnp.int32), jnp.zeros((_L,), jnp.float32)))
            pc = plsc.cumsum(acc) + cum
            psc = plsc.cumsum(accs) + cums
            p_ref[pl.ds(c * _L, _L)] = pc
            ps_ref[pl.ds(c * _L, _L)] = psc
            cum = cum + jnp.sum(acc)
            cums = cums + jnp.sum(accs)
        return cum, cums

    def find_cross(thresh):
        found = jnp.int32(0)
        b_star = jnp.int32(0)
        p_star = jnp.int32(0)
        for c in range(_NBIN // _L):
            pc = p_ref[pl.ds(c * _L, _L)]
            m = pc > thresh
            cand = jnp.where(m, 255 - (iota + c * _L), -1)
            mx = jnp.max(cand)
            bloc = 255 - mx
            pmin = -jnp.max(jnp.where(m, -pc, i32min))
            any_m = mx >= 0
            take = (found == 0) & any_m
            b_star = jnp.where(take, bloc, b_star)
            p_star = jnp.where(take, pmin, p_star)
            found = jnp.where(any_m, jnp.int32(1), found)
        p_prev = jnp.int32(0)
        s_at = jnp.float32(0.0)
        s_prev = jnp.float32(0.0)
        for c in range(_NBIN // _L):
            bins = iota + c * _L
            pc = p_ref[pl.ds(c * _L, _L)]
            psc = ps_ref[pl.ds(c * _L, _L)]
            p_prev = p_prev + jnp.sum(jnp.where(bins == b_star - 1, pc, 0))
            s_at = s_at + jnp.sum(jnp.where(bins == b_star, psc, 0.0))
            s_prev = s_prev + jnp.sum(jnp.where(bins == b_star - 1, psc, 0.0))
        return b_star, p_star, p_prev, s_at, s_prev

    def stream_pass(row, chunk_fn, load_x=False):
        cps = [None, None]
        xcps = [None, None]
        cps[0] = pltpu.async_copy(
            t_hbm.at[row, pl.ds(0, swords)], tring.at[0], sems[0])
        if load_x:
            xcps[0] = pltpu.async_copy(
                x_hbm.at[row, pl.ds(0, swords)],
                xv.at[pl.ds(0, swords)], xsems[0])
        for s in range(_NSEC):
            par = s % 2
            if s + 1 < _NSEC:
                parn = (s + 1) % 2
                cps[parn] = pltpu.async_copy(
                    t_hbm.at[row, pl.ds((s + 1) * swords, swords)],
                    tring.at[parn], sems[parn])
                if load_x:
                    xcps[parn] = pltpu.async_copy(
                        x_hbm.at[row, pl.ds((s + 1) * swords, swords)],
                        xv.at[pl.ds((s + 1) * swords, swords)], xsems[parn])
            cps[par].wait()
            if load_x:
                xcps[par].wait()

            @plsc.parallel_loop(0, schunk, unroll=4)
            def _cbody(ci):
                v = xv[pl.ds((s * schunk + ci) * _L, _L)]
                tval = tring[par, pl.ds(ci * _L, _L)]
                d = v - tval
                chunk_fn(s * schunk + ci, v, d * d)

    def row_body(r, _):
        row = roff + jnp.minimum(wid * rows_per + r, nsc - 1)
        zero_hists()

        def p1_chunk(ci, v, d2):
            key = keys_of(v)
            bin1 = lax.shift_right_arithmetic(key, 24) + 128
            bank = (ci & (_NBANK - 1)) * (_NBIN * _L)
            addr = bank + lane_base + bin1
            plsc.addupdate_scatter(histc, [addr], ones)
            plsc.addupdate_scatter(hists, [addr], d2)
        stream_pass(row, p1_chunk, load_x=True)

        n_tot, s_tot1 = build_p()
        b1, p1, _pp, s1_at, _sp = find_cross(jnp.int32(n - kk))
        g8 = jnp.int32(n) - p1
        s_hi1 = s_tot1 - s1_at

        zero_hists()

        def p2_chunk(ci, v, d2):
            key = keys_of(v)
            bin1 = lax.shift_right_arithmetic(key, 24) + 128
            bin2 = lax.shift_right_arithmetic(key, 16) & 0xFF
            m = bin1 == b1
            bank = (ci & (_NBANK - 1)) * (_NBIN * _L)
            addr = bank + lane_base + bin2
            plsc.addupdate_scatter(histc, [addr], ones, mask=m)
            plsc.addupdate_scatter(hists, [addr], d2, mask=m)
        stream_pass(row, p2_chunk)

        e8, s_tot2 = build_p()
        b2, p2, pprev2, s2_at, s2_prev = find_cross(g8 + e8 - jnp.int32(kk))
        g16 = g8 + (e8 - p2)
        e16 = p2 - pprev2
        s_hi = s_hi1 + (s_tot2 - s2_at)
        s_band = s2_at - s2_prev

        vec = (jnp.where(iota == 0, s_hi, 0.0)
               + jnp.where(iota == 1, s_band, 0.0)
               + jnp.where(iota == 2, g16.astype(jnp.float32), 0.0)
               + jnp.where(iota == 3, e16.astype(jnp.float32), 0.0))
        outv[pl.ds(r * _L, _L)] = vec
        return 0

    lax.fori_loop(0, rows_per, row_body, 0)
    pltpu.sync_copy(outv, out_hbm.at[wid])


def _sc_spatial(x2d, t2d, kk, roff):
    nrows, n = x2d.shape
    nsc = nrows - roff
    rows_per = (nsc + _NSUB - 1) // _NSUB
    mesh = plsc.VectorSubcoreMesh(core_axis_name="c", subcore_axis_name="s")
    body = functools.partial(_sc_body, nsc, roff, n, kk, rows_per)
    f = pl.kernel(
        body,
        mesh=mesh,
        compiler_params=pltpu.CompilerParams(needs_layout_passes=False),
        out_type=jax.ShapeDtypeStruct((_NSUB, rows_per * _L), jnp.float32),
        scratch_types=[
            pltpu.VMEM((n,), jnp.float32),
            pltpu.VMEM((2, n // _NSEC), jnp.float32),
            pltpu.VMEM((_HWORDS,), jnp.int32),
            pltpu.VMEM((_HWORDS,), jnp.float32),
            pltpu.VMEM((_NBIN,), jnp.int32),
            pltpu.VMEM((_NBIN,), jnp.float32),
            pltpu.VMEM((rows_per * _L,), jnp.float32),
            pltpu.SemaphoreType.DMA,
            pltpu.SemaphoreType.DMA,
            pltpu.SemaphoreType.DMA,
            pltpu.SemaphoreType.DMA,
        ],
    )
    return f(x2d, t2d)


def kernel(output, target):
    B, T, C, H, W = output.shape
    hw = H * W
    kk = hw // 10
    nrows = B * T * C
    nblocks = B * (T // _WIN)
    nsplit = (nblocks * 7) // 16
    roff = nsplit * _WIN * C
    xs = output.reshape(nrows, hw)
    ts = target.reshape(nrows, hw)
    sc_part = _sc_spatial(xs, ts, kk, roff)
    time_sse, tc_spatial = _tc_part(output, target, nsplit, kk)
    rows_per = sc_part.shape[1] // _L
    p = sc_part.reshape(_NSUB * rows_per, _L)[:nrows - roff]
    s_hi, s_band, g, e = p[:, 0], p[:, 1], p[:, 2], p[:, 3]
    spatial_sum = tc_spatial + jnp.sum(s_hi + (kk - g) / e * s_band)
    tnorm = jnp.float32(B * C * hw * (T // _WIN))
    snorm = jnp.float32(nrows * kk)
    return time_sse / tnorm + spatial_sum / snorm


# speedup vs baseline: 1.3093x; 1.0272x over previous
import functools

import jax
import jax.numpy as jnp
from jax import lax
from jax.experimental import pallas as pl
from jax.experimental.pallas import tpu as pltpu
from jax.experimental.pallas import tpu_sc as plsc

_WIN = 4
_LANE = 128
_L = 16
_NSUB = 32
_NBIN = 256
_NSEC = 8


def _tc_kernel(nsplit, kk, x_ref, t_ref, out_ref, acc_ref):
    step = pl.program_id(0)
    x = x_ref[0]
    tg = t_ref[0]
    nc = x.shape[1]

    @pl.when(step == 0)
    def _():
        acc_ref[0] = 0.0
        acc_ref[1] = 0.0

    mo = jnp.maximum(jnp.maximum(x[0], x[1]), jnp.maximum(x[2], x[3]))
    mt = jnp.maximum(jnp.maximum(tg[0], tg[1]), jnp.maximum(tg[2], tg[3]))
    dt = mo - mt
    acc_ref[0] = acc_ref[0] + jnp.sum(dt * dt)

    @pl.when(step < nsplit)
    def _():
        bits = lax.bitcast_convert_type(x, jnp.int32)
        key = jnp.where(bits < 0, bits ^ jnp.int32(0x7FFFFFFF), bits)
        key16 = lax.shift_right_arithmetic(key, 16)

        def body(i, lohi):
            lo, hi = lohi
            mid = lax.shift_right_arithmetic(lo + hi, 1)
            cnt = jnp.sum((key16 > mid).astype(jnp.int32), axis=(2, 3),
                          keepdims=True)
            pred = cnt < kk
            return jnp.where(pred, lo, mid), jnp.where(pred, mid, hi)

        lo0 = jnp.full((_WIN, nc, 1, 1), -32769, jnp.int32)
        hi0 = jnp.full((_WIN, nc, 1, 1), 32767, jnp.int32)
        _, hi = lax.fori_loop(0, 16, body, (lo0, hi0))

        d2 = (x - tg) * (x - tg)
        mhi = key16 > hi
        mband = key16 == hi
        s_hi = jnp.sum(jnp.where(mhi, d2, 0.0))
        s_band = jnp.sum(jnp.where(mband, d2, 0.0), axis=(2, 3),
                         keepdims=True)
        g = jnp.sum(mhi.astype(jnp.float32), axis=(2, 3), keepdims=True)
        e = jnp.sum(mband.astype(jnp.float32), axis=(2, 3), keepdims=True)
        w = (jnp.float32(kk) - g) / e
        acc_ref[1] = acc_ref[1] + s_hi + jnp.sum(w * s_band)

    @pl.when(step == pl.num_programs(0) - 1)
    def _():
        out_ref[0, 0] = acc_ref[0]
        out_ref[0, 1] = acc_ref[1]


def _tc_part(output, target, nsplit, kk):
    B, T, C, H, W = output.shape
    hw = H * W
    sub = hw // _LANE
    nw = T // _WIN
    xr = output.reshape(B * nw, _WIN, C, sub, _LANE)
    tr = target.reshape(B * nw, _WIN, C, sub, _LANE)
    spec = pl.BlockSpec((1, _WIN, C, sub, _LANE),
                        lambda r: (r, 0, 0, 0, 0))
    out = pl.pallas_call(
        functools.partial(_tc_kernel, nsplit, kk),
        grid=(B * nw,),
        in_specs=[spec, spec],
        out_specs=pl.BlockSpec(memory_space=pltpu.SMEM),
        out_shape=jax.ShapeDtypeStruct((1, 2), jnp.float32),
        scratch_shapes=[pltpu.SMEM((2,), jnp.float32)],
    )(xr, tr)
    return out[0, 0], out[0, 1]


_NBANK = 1
_NCOPY = _NBANK * _L
_HWORDS = _NBIN * _NCOPY


def _sc_body(nsc, roff, n, kk, rows_per, x_hbm, t_hbm, out_hbm,
             xv, tring, histc, hists, p_ref, ps_ref, outv,
             sem0, sem1, sem2, sem3):
    cid = lax.axis_index("c")
    sid = lax.axis_index("s")
    wid = sid * 2 + cid

    iota = lax.iota(jnp.int32, _L)
    lane_base = iota * _NBIN
    ones = jnp.ones((_L,), jnp.int32)
    nchunk = n // _L
    swords = n // _NSEC
    schunk = swords // _L
    i32min = jnp.int32(-2147483648)
    sems = (sem0, sem1)
    xsems = (sem2, sem3)

    def zero_hists():
        @plsc.parallel_loop(0, _HWORDS // _L, unroll=4)
        def _z(i):
            histc[pl.ds(i * _L, _L)] = jnp.zeros((_L,), jnp.int32)
            hists[pl.ds(i * _L, _L)] = jnp.zeros((_L,), jnp.float32)

    def keys_of(v):
        bits = lax.bitcast_convert_type(v, jnp.int32)
        return jnp.where(bits < 0, bits ^ jnp.int32(0x7FFFFFFF), bits)

    def build_p():
        cum = jnp.int32(0)
        cums = jnp.float32(0.0)
        for c in range(_NBIN // _L):
            def mbody(j, acc):
                a, s = acc
                a = a + histc[pl.ds(j * _NBIN + c * _L, _L)]
                s = s + hists[pl.ds(j * _NBIN + c * _L, _L)]
                return a, s
            acc, accs = lax.fori_loop(
                0, _NCOPY, mbody,
                (jnp.zeros((_L,), jnp.int32), jnp.zeros((_L,), jnp.float32)))
            pc = plsc.cumsum(acc) + cum
            psc = plsc.cumsum(accs) + cums
            p_ref[pl.ds(c * _L, _L)] = pc
            ps_ref[pl.ds(c * _L, _L)] = psc
            cum = cum + jnp.sum(acc)
            cums = cums + jnp.sum(accs)
        return cum, cums

    def find_cross(thresh):
        found = jnp.int32(0)
        b_star = jnp.int32(0)
        p_star = jnp.int32(0)
        for c in range(_NBIN // _L):
            pc = p_ref[pl.ds(c * _L, _L)]
            m = pc > thresh
            cand = jnp.where(m, 255 - (iota + c * _L), -1)
            mx = jnp.max(cand)
            bloc = 255 - mx
            pmin = -jnp.max(jnp.where(m, -pc, i32min))
            any_m = mx >= 0
            take = (found == 0) & any_m
            b_star = jnp.where(take, bloc, b_star)
            p_star = jnp.where(take, pmin, p_star)
            found = jnp.where(any_m, jnp.int32(1), found)
        p_prev = jnp.int32(0)
        s_at = jnp.float32(0.0)
        s_prev = jnp.float32(0.0)
        for c in range(_NBIN // _L):
            bins = iota + c * _L
            pc = p_ref[pl.ds(c * _L, _L)]
            psc = ps_ref[pl.ds(c * _L, _L)]
            p_prev = p_prev + jnp.sum(jnp.where(bins == b_star - 1, pc, 0))
            s_at = s_at + jnp.sum(jnp.where(bins == b_star, psc, 0.0))
            s_prev = s_prev + jnp.sum(jnp.where(bins == b_star - 1, psc, 0.0))
        return b_star, p_star, p_prev, s_at, s_prev

    def stream_pass(row, chunk_fn, load_x=False):
        cps = [None, None]
        xcps = [None, None]
        cps[0] = pltpu.async_copy(
            t_hbm.at[row, pl.ds(0, swords)], tring.at[0], sems[0])
        if load_x:
            xcps[0] = pltpu.async_copy(
                x_hbm.at[row, pl.ds(0, swords)],
                xv.at[pl.ds(0, swords)], xsems[0])
        for s in range(_NSEC):
            par = s % 2
            if s + 1 < _NSEC:
                parn = (s + 1) % 2
                cps[parn] = pltpu.async_copy(
                    t_hbm.at[row, pl.ds((s + 1) * swords, swords)],
                    tring.at[parn], sems[parn])
                if load_x:
                    xcps[parn] = pltpu.async_copy(
                        x_hbm.at[row, pl.ds((s + 1) * swords, swords)],
                        xv.at[pl.ds((s + 1) * swords, swords)], xsems[parn])
            cps[par].wait()
            if load_x:
                xcps[par].wait()

            @plsc.parallel_loop(0, schunk, unroll=4)
            def _cbody(ci):
                v = xv[pl.ds((s * schunk + ci) * _L, _L)]
                tval = tring[par, pl.ds(ci * _L, _L)]
                d = v - tval
                chunk_fn(s * schunk + ci, v, d * d)

    def row_body(r, _):
        row = roff + jnp.minimum(wid * rows_per + r, nsc - 1)
        zero_hists()

        def p1_chunk(ci, v, d2):
            key = keys_of(v)
            bin1 = lax.shift_right_arithmetic(key, 24) + 128
            bank = (ci & (_NBANK - 1)) * (_NBIN * _L)
            addr = bank + lane_base + bin1
            plsc.addupdate_scatter(histc, [addr], ones)
            plsc.addupdate_scatter(hists, [addr], d2)
        stream_pass(row, p1_chunk, load_x=True)

        n_tot, s_tot1 = build_p()
        b1, p1, _pp, s1_at, _sp = find_cross(jnp.int32(n - kk))
        g8 = jnp.int32(n) - p1
        s_hi1 = s_tot1 - s1_at

        zero_hists()

        def p2_chunk(ci, v, d2):
            key = keys_of(v)
            bin1 = lax.shift_right_arithmetic(key, 24) + 128
            bin2 = lax.shift_right_arithmetic(key, 16) & 0xFF
            m = bin1 == b1
            bank = (ci & (_NBANK - 1)) * (_NBIN * _L)
            addr = bank + lane_base + bin2
            plsc.addupdate_scatter(histc, [addr], ones, mask=m)
            plsc.addupdate_scatter(hists, [addr], d2, mask=m)
        stream_pass(row, p2_chunk)

        e8, s_tot2 = build_p()
        b2, p2, pprev2, s2_at, s2_prev = find_cross(g8 + e8 - jnp.int32(kk))
        g16 = g8 + (e8 - p2)
        e16 = p2 - pprev2
        s_hi = s_hi1 + (s_tot2 - s2_at)
        s_band = s2_at - s2_prev

        vec = (jnp.where(iota == 0, s_hi, 0.0)
               + jnp.where(iota == 1, s_band, 0.0)
               + jnp.where(iota == 2, g16.astype(jnp.float32), 0.0)
               + jnp.where(iota == 3, e16.astype(jnp.float32), 0.0))
        outv[pl.ds(r * _L, _L)] = vec
        return 0

    lax.fori_loop(0, rows_per, row_body, 0)
    pltpu.sync_copy(outv, out_hbm.at[wid])


def _sc_spatial(x2d, t2d, kk, roff):
    nrows, n = x2d.shape
    nsc = nrows - roff
    rows_per = (nsc + _NSUB - 1) // _NSUB
    mesh = plsc.VectorSubcoreMesh(core_axis_name="c", subcore_axis_name="s")
    body = functools.partial(_sc_body, nsc, roff, n, kk, rows_per)
    f = pl.kernel(
        body,
        mesh=mesh,
        compiler_params=pltpu.CompilerParams(needs_layout_passes=False),
        out_type=jax.ShapeDtypeStruct((_NSUB, rows_per * _L), jnp.float32),
        scratch_types=[
            pltpu.VMEM((n,), jnp.float32),
            pltpu.VMEM((2, n // _NSEC), jnp.float32),
            pltpu.VMEM((_HWORDS,), jnp.int32),
            pltpu.VMEM((_HWORDS,), jnp.float32),
            pltpu.VMEM((_NBIN,), jnp.int32),
            pltpu.VMEM((_NBIN,), jnp.float32),
            pltpu.VMEM((rows_per * _L,), jnp.float32),
            pltpu.SemaphoreType.DMA,
            pltpu.SemaphoreType.DMA,
            pltpu.SemaphoreType.DMA,
            pltpu.SemaphoreType.DMA,
        ],
    )
    return f(x2d, t2d)


def kernel(output, target):
    B, T, C, H, W = output.shape
    hw = H * W
    kk = hw // 10
    nrows = B * T * C
    nblocks = B * (T // _WIN)
    nsplit = (nblocks * 6) // 16
    roff = nsplit * _WIN * C
    xs = output.reshape(nrows, hw)
    ts = target.reshape(nrows, hw)
    sc_part = _sc_spatial(xs, ts, kk, roff)
    time_sse, tc_spatial = _tc_part(output, target, nsplit, kk)
    rows_per = sc_part.shape[1] // _L
    p = sc_part.reshape(_NSUB * rows_per, _L)[:nrows - roff]
    s_hi, s_band, g, e = p[:, 0], p[:, 1], p[:, 2], p[:, 3]
    spatial_sum = tc_spatial + jnp.sum(s_hi + (kk - g) / e * s_band)
    tnorm = jnp.float32(B * C * hw * (T // _WIN))
    snorm = jnp.float32(nrows * kk)
    return time_sse / tnorm + spatial_sum / snorm
